# baseline (speedup 1.0000x reference)
import functools

import jax
import jax.numpy as jnp
from jax import lax
from jax.experimental import pallas as pl
from jax.experimental.pallas import tpu as pltpu
from jax.experimental.pallas import tpu_sc as plsc

VOCAB = 100000
D_MODEL = 128
BATCH = 1024

BN = 2048


def _make_sc_gather(V, D, B):
    info = plsc.get_sparse_core_info()
    NC, NS = info.num_cores, info.num_subcores
    NW = NC * NS
    b_per_w = B // NW
    mesh = plsc.VectorSubcoreMesh(core_axis_name="c", subcore_axis_name="s")

    @functools.partial(
        pl.kernel,
        mesh=mesh,
        out_type=jax.ShapeDtypeStruct((B, D), jnp.float32),
        scratch_types=[
            pltpu.VMEM((b_per_w,), jnp.int32),
            pltpu.VMEM((b_per_w, D), jnp.float32),
            pltpu.SemaphoreType.DMA,
        ],
    )
    def gather(table_hbm, idx_hbm, out_hbm, idx_v, rows_v, sem):
        wid = lax.axis_index("s") * NC + lax.axis_index("c")
        base = wid * b_per_w
        pltpu.sync_copy(idx_hbm.at[pl.ds(base, b_per_w)], idx_v)
        pltpu.async_copy(table_hbm.at[idx_v], rows_v, sem).wait()
        pltpu.sync_copy(rows_v, out_hbm.at[pl.ds(base, b_per_w)])

    return gather


def _mm_body(x_ref, w_ref, b_ref, o_ref):
    o_ref[...] = (
        lax.dot_general(
            x_ref[...],
            w_ref[...],
            (((1,), (1,)), ((), ())),
            preferred_element_type=jnp.float32,
        )
        + b_ref[...]
    )


def _tc_project(x, w, b2d):
    return pl.pallas_call(
        _mm_body,
        grid=(pl.cdiv(VOCAB, BN),),
        in_specs=[
            pl.BlockSpec((BATCH, D_MODEL), lambda j: (0, 0)),
            pl.BlockSpec((BN, D_MODEL), lambda j: (j, 0)),
            pl.BlockSpec((1, BN), lambda j: (0, j)),
        ],
        out_specs=pl.BlockSpec((BATCH, BN), lambda j: (0, j)),
        out_shape=jax.ShapeDtypeStruct((BATCH, VOCAB), jnp.float32),
    )(x, w, b2d)


def kernel(input_ids, embed_table, proj_w, proj_b):
    x = jnp.take(embed_table, input_ids, axis=0)
    return _tc_project(x, proj_w, proj_b.reshape(1, VOCAB))


# speedup vs baseline: 3.1455x; 3.1455x over previous
import functools

import jax
import jax.numpy as jnp
from jax import lax
from jax.experimental import pallas as pl
from jax.experimental.pallas import tpu as pltpu
from jax.experimental.pallas import tpu_sc as plsc

VOCAB = 100000
D_MODEL = 128
BATCH = 1024

BN = 2048


def _make_sc_gather(V, D, B):
    info = plsc.get_sparse_core_info()
    NC, NS = info.num_cores, info.num_subcores
    NW = NC * NS
    b_per_w = B // NW
    mesh = plsc.VectorSubcoreMesh(core_axis_name="c", subcore_axis_name="s")

    @functools.partial(
        pl.kernel,
        mesh=mesh,
        out_type=jax.ShapeDtypeStruct((B, D), jnp.float32),
        scratch_types=[
            pltpu.VMEM((b_per_w,), jnp.int32),
            pltpu.VMEM((b_per_w, D), jnp.float32),
            pltpu.SemaphoreType.DMA,
        ],
    )
    def gather(table_hbm, idx_hbm, out_hbm, idx_v, rows_v, sem):
        wid = lax.axis_index("s") * NC + lax.axis_index("c")
        base = wid * b_per_w
        pltpu.sync_copy(idx_hbm.at[pl.ds(base, b_per_w)], idx_v)
        pltpu.async_copy(table_hbm.at[idx_v], rows_v, sem).wait()
        pltpu.sync_copy(rows_v, out_hbm.at[pl.ds(base, b_per_w)])

    return gather


def _mm_body(w_ref, x_ref, b_ref, o_ref):
    acc = lax.dot_general(
        w_ref[...],
        x_ref[...],
        (((1,), (1,)), ((), ())),
        preferred_element_type=jnp.float32,
    )
    bcol = lax.dot_general(
        b_ref[...],
        jnp.ones((1, 1), jnp.float32),
        (((0,), (0,)), ((), ())),
        preferred_element_type=jnp.float32,
    )
    o_ref[...] = acc + bcol


def _tc_project_t(x, w, b2d):
    return pl.pallas_call(
        _mm_body,
        grid=(pl.cdiv(VOCAB, BN),),
        in_specs=[
            pl.BlockSpec((BN, D_MODEL), lambda j: (j, 0)),
            pl.BlockSpec((BATCH, D_MODEL), lambda j: (0, 0)),
            pl.BlockSpec((1, BN), lambda j: (0, j)),
        ],
        out_specs=pl.BlockSpec((BN, BATCH), lambda j: (j, 0)),
        out_shape=jax.ShapeDtypeStruct((VOCAB, BATCH), jnp.float32),
    )(w, x, b2d)


def kernel(input_ids, embed_table, proj_w, proj_b):
    gather = _make_sc_gather(VOCAB, D_MODEL, BATCH)
    x = gather(embed_table, input_ids.astype(jnp.int32))
    logits_t = _tc_project_t(x, proj_w, proj_b.reshape(1, VOCAB))
    return logits_t.T


# speedup vs baseline: 3.2079x; 1.0198x over previous
import functools

import jax
import jax.numpy as jnp
from jax import lax
from jax.experimental import pallas as pl
from jax.experimental.pallas import tpu as pltpu
from jax.experimental.pallas import tpu_sc as plsc

VOCAB = 100000
D_MODEL = 128
BATCH = 1024

BN = 4096


def _make_sc_gather(V, D, B):
    info = plsc.get_sparse_core_info()
    NC, NS = info.num_cores, info.num_subcores
    NW = NC * NS
    b_per_w = B // NW
    mesh = plsc.VectorSubcoreMesh(core_axis_name="c", subcore_axis_name="s")

    @functools.partial(
        pl.kernel,
        mesh=mesh,
        out_type=jax.ShapeDtypeStruct((B, D), jnp.float32),
        scratch_types=[
            pltpu.VMEM((b_per_w,), jnp.int32),
            pltpu.VMEM((b_per_w, D), jnp.float32),
            pltpu.SemaphoreType.DMA,
        ],
    )
    def gather(table_hbm, idx_hbm, out_hbm, idx_v, rows_v, sem):
        wid = lax.axis_index("s") * NC + lax.axis_index("c")
        base = wid * b_per_w
        pltpu.sync_copy(idx_hbm.at[pl.ds(base, b_per_w)], idx_v)
        pltpu.async_copy(table_hbm.at[idx_v], rows_v, sem).wait()
        pltpu.sync_copy(rows_v, out_hbm.at[pl.ds(base, b_per_w)])

    return gather


def _mm_body(w_ref, x_ref, b_ref, o_ref):
    acc = lax.dot_general(
        w_ref[...],
        x_ref[...],
        (((1,), (1,)), ((), ())),
        preferred_element_type=jnp.float32,
    )
    bcol = lax.dot_general(
        b_ref[...],
        jnp.ones((1, 1), jnp.float32),
        (((0,), (0,)), ((), ())),
        preferred_element_type=jnp.float32,
    )
    o_ref[...] = acc + bcol


def _tc_project_t(x, w, b2d):
    return pl.pallas_call(
        _mm_body,
        grid=(pl.cdiv(VOCAB, BN),),
        in_specs=[
            pl.BlockSpec((BN, D_MODEL), lambda j: (j, 0)),
            pl.BlockSpec((BATCH, D_MODEL), lambda j: (0, 0)),
            pl.BlockSpec((1, BN), lambda j: (0, j)),
        ],
        out_specs=pl.BlockSpec((BN, BATCH), lambda j: (j, 0)),
        out_shape=jax.ShapeDtypeStruct((VOCAB, BATCH), jnp.float32),
    )(w, x, b2d)


def kernel(input_ids, embed_table, proj_w, proj_b):
    gather = _make_sc_gather(VOCAB, D_MODEL, BATCH)
    x = gather(embed_table, input_ids.astype(jnp.int32))
    logits_t = _tc_project_t(x, proj_w, proj_b.reshape(1, VOCAB))
    return logits_t.T


# speedup vs baseline: 3.2149x; 1.0022x over previous
import functools

import jax
import jax.numpy as jnp
from jax import lax
from jax.experimental import pallas as pl
from jax.experimental.pallas import tpu as pltpu
from jax.experimental.pallas import tpu_sc as plsc

VOCAB = 100000
D_MODEL = 128
BATCH = 1024

BN = 6144


def _make_sc_gather(V, D, B):
    info = plsc.get_sparse_core_info()
    NC, NS = info.num_cores, info.num_subcores
    NW = NC * NS
    b_per_w = B // NW
    mesh = plsc.VectorSubcoreMesh(core_axis_name="c", subcore_axis_name="s")

    @functools.partial(
        pl.kernel,
        mesh=mesh,
        out_type=jax.ShapeDtypeStruct((B, D), jnp.float32),
        scratch_types=[
            pltpu.VMEM((b_per_w,), jnp.int32),
            pltpu.VMEM((b_per_w, D), jnp.float32),
            pltpu.SemaphoreType.DMA,
        ],
    )
    def gather(table_hbm, idx_hbm, out_hbm, idx_v, rows_v, sem):
        wid = lax.axis_index("s") * NC + lax.axis_index("c")
        base = wid * b_per_w
        pltpu.sync_copy(idx_hbm.at[pl.ds(base, b_per_w)], idx_v)
        pltpu.async_copy(table_hbm.at[idx_v], rows_v, sem).wait()
        pltpu.sync_copy(rows_v, out_hbm.at[pl.ds(base, b_per_w)])

    return gather


def _mm_body(w_ref, x_ref, b_ref, o_ref):
    acc = lax.dot_general(
        w_ref[...],
        x_ref[...],
        (((1,), (1,)), ((), ())),
        preferred_element_type=jnp.float32,
    )
    bcol = lax.dot_general(
        b_ref[...],
        jnp.ones((1, 1), jnp.float32),
        (((0,), (0,)), ((), ())),
        preferred_element_type=jnp.float32,
    )
    o_ref[...] = acc + bcol


def _tc_project_t(x, w, b2d):
    return pl.pallas_call(
        _mm_body,
        grid=(pl.cdiv(VOCAB, BN),),
        in_specs=[
            pl.BlockSpec((BN, D_MODEL), lambda j: (j, 0)),
            pl.BlockSpec((BATCH, D_MODEL), lambda j: (0, 0)),
            pl.BlockSpec((1, BN), lambda j: (0, j)),
        ],
        out_specs=pl.BlockSpec((BN, BATCH), lambda j: (j, 0)),
        out_shape=jax.ShapeDtypeStruct((VOCAB, BATCH), jnp.float32),
    )(w, x, b2d)


def kernel(input_ids, embed_table, proj_w, proj_b):
    gather = _make_sc_gather(VOCAB, D_MODEL, BATCH)
    x = gather(embed_table, input_ids.astype(jnp.int32))
    logits_t = _tc_project_t(x, proj_w, proj_b.reshape(1, VOCAB))
    return logits_t.T


# speedup vs baseline: 3.2327x; 1.0056x over previous
import functools

import jax
import jax.numpy as jnp
from jax import lax
from jax.experimental import pallas as pl
from jax.experimental.pallas import tpu as pltpu
from jax.experimental.pallas import tpu_sc as plsc

VOCAB = 100000
D_MODEL = 128
BATCH = 1024

BN = 6144


def _make_sc_gather(V, D, B):
    info = plsc.get_sparse_core_info()
    NC, NS = 1, info.num_subcores
    NW = NC * NS
    b_per_w = B // NW
    mesh = plsc.VectorSubcoreMesh(
        core_axis_name="c", subcore_axis_name="s", num_cores=1
    )

    @functools.partial(
        pl.kernel,
        mesh=mesh,
        out_type=jax.ShapeDtypeStruct((B, D), jnp.float32),
        scratch_types=[
            pltpu.VMEM((b_per_w,), jnp.int32),
            pltpu.VMEM((b_per_w, D), jnp.float32),
            pltpu.SemaphoreType.DMA,
        ],
    )
    def gather(table_hbm, idx_hbm, out_hbm, idx_v, rows_v, sem):
        wid = lax.axis_index("s") * NC + lax.axis_index("c")
        base = wid * b_per_w
        pltpu.sync_copy(idx_hbm.at[pl.ds(base, b_per_w)], idx_v)
        pltpu.async_copy(table_hbm.at[idx_v], rows_v, sem).wait()
        pltpu.sync_copy(rows_v, out_hbm.at[pl.ds(base, b_per_w)])

    return gather


def _mm_body(w_ref, x_ref, b_ref, o_ref):
    acc = lax.dot_general(
        w_ref[...],
        x_ref[...],
        (((1,), (1,)), ((), ())),
        preferred_element_type=jnp.float32,
    )
    bcol = lax.dot_general(
        b_ref[...],
        jnp.ones((1, 1), jnp.float32),
        (((0,), (0,)), ((), ())),
        preferred_element_type=jnp.float32,
    )
    o_ref[...] = acc + bcol


def _tc_project_t(x, w, b2d):
    return pl.pallas_call(
        _mm_body,
        grid=(pl.cdiv(VOCAB, BN),),
        in_specs=[
            pl.BlockSpec((BN, D_MODEL), lambda j: (j, 0)),
            pl.BlockSpec((BATCH, D_MODEL), lambda j: (0, 0)),
            pl.BlockSpec((1, BN), lambda j: (0, j)),
        ],
        out_specs=pl.BlockSpec((BN, BATCH), lambda j: (j, 0)),
        out_shape=jax.ShapeDtypeStruct((VOCAB, BATCH), jnp.float32),
        compiler_params=pltpu.CompilerParams(vmem_limit_bytes=100 * 1024 * 1024),
    )(w, x, b2d)


def kernel(input_ids, embed_table, proj_w, proj_b):
    gather = _make_sc_gather(VOCAB, D_MODEL, BATCH)
    x = gather(embed_table, input_ids.astype(jnp.int32))
    logits_t = _tc_project_t(x, proj_w, proj_b.reshape(1, VOCAB))
    return logits_t.T
